# Initial kernel scaffold; baseline (speedup 1.0000x reference)
import jax
import jax.numpy as jnp
from jax.experimental import pallas as pl


def kernel(features, edge_index, W1, b1, W2, b2, Wf, bf):
    raise NotImplementedError("write your pallas kernel here")


# speedup vs baseline: 6.0472x; 6.0472x over previous
import functools

import jax
import jax.numpy as jnp
from jax import lax
from jax.experimental import pallas as pl
from jax.experimental.pallas import tpu as pltpu
from jax.experimental.pallas import tpu_sc as plsc

N = 10000
NP = 10240
E = 320000
CHUNK = 128
NCORES = 2
NTILES = 16
NWORK = NCORES * NTILES
EPAD = ((E + NWORK * CHUNK - 1) // (NWORK * CHUNK)) * (NWORK * CHUNK)
EDGES_PER_TILE = EPAD // NWORK
NCHUNK = EDGES_PER_TILE // CHUNK
STRIPE = NP // NTILES
D_IN = 128
DH = 64
DO = 40
DW = 128

_mesh = plsc.VectorSubcoreMesh(core_axis_name="c", subcore_axis_name="s")


@functools.partial(
    pl.kernel,
    out_type=[jax.ShapeDtypeStruct((NCORES * NP,), jnp.float32),
              jax.ShapeDtypeStruct((NCORES * NP,), jnp.float32)],
    mesh=_mesh,
    scratch_types=[
        pltpu.VMEM((CHUNK,), jnp.int32),
        pltpu.VMEM((CHUNK,), jnp.int32),
        pltpu.VMEM((CHUNK,), jnp.float32),
        pltpu.VMEM((STRIPE,), jnp.float32),
        pltpu.VMEM_SHARED((NP,), jnp.float32),
        pltpu.VMEM_SHARED((NP,), jnp.float32),
    ],
)
def _hist_kernel(src_hbm, dst_hbm, hs_hbm, hd_hbm,
                 srcv, dstv, onesv, zbuf, acc_s, acc_d):
    c = lax.axis_index("c")
    s = lax.axis_index("s")
    z16 = jnp.zeros((16,), jnp.float32)
    o16 = jnp.ones((16,), jnp.float32)
    for j in range(CHUNK // 16):
        onesv[pl.ds(j * 16, 16)] = o16
    for j in range(STRIPE // 16):
        zbuf[pl.ds(j * 16, 16)] = z16
    pltpu.sync_copy(zbuf, acc_s.at[pl.ds(s * STRIPE, STRIPE)])
    pltpu.sync_copy(zbuf, acc_d.at[pl.ds(s * STRIPE, STRIPE)])
    plsc.subcore_barrier()

    base0 = (c * NTILES + s) * EDGES_PER_TILE

    def body(k, carry):
        base = base0 + k * CHUNK
        pltpu.sync_copy(src_hbm.at[pl.ds(base, CHUNK)], srcv)
        pltpu.sync_copy(dst_hbm.at[pl.ds(base, CHUNK)], dstv)
        pltpu.sync_copy(onesv, acc_s.at[srcv], add=True)
        pltpu.sync_copy(onesv, acc_d.at[dstv], add=True)
        return carry

    lax.fori_loop(0, NCHUNK, body, 0)
    plsc.subcore_barrier()
    off = c * NP + s * STRIPE
    pltpu.sync_copy(acc_s.at[pl.ds(s * STRIPE, STRIPE)], hs_hbm.at[pl.ds(off, STRIPE)])
    pltpu.sync_copy(acc_d.at[pl.ds(s * STRIPE, STRIPE)], hd_hbm.at[pl.ds(off, STRIPE)])


@functools.partial(
    pl.kernel,
    out_type=jax.ShapeDtypeStruct((NCORES * NP, DW), jnp.float32),
    mesh=_mesh,
    scratch_types=[
        pltpu.VMEM((CHUNK,), jnp.int32),
        pltpu.VMEM((CHUNK,), jnp.int32),
        pltpu.VMEM((CHUNK, DW), jnp.float32),
        pltpu.VMEM_SHARED((NP, DW), jnp.float32),
        pltpu.SemaphoreType.DMA,
    ],
)
def _edge_pass(ytab_hbm, src_hbm, dst_hbm, zer_hbm, agg_hbm,
               srcv, dstv, rows, acc, gsem):
    c = lax.axis_index("c")
    s = lax.axis_index("s")

    pltpu.sync_copy(zer_hbm, acc.at[pl.ds(s * STRIPE, STRIPE), :])
    plsc.subcore_barrier()

    base0 = (c * NTILES + s) * EDGES_PER_TILE

    def body(k, carry):
        base = base0 + k * CHUNK
        pltpu.sync_copy(src_hbm.at[pl.ds(base, CHUNK)], srcv)
        pltpu.sync_copy(dst_hbm.at[pl.ds(base, CHUNK)], dstv)
        pltpu.async_copy(ytab_hbm.at[srcv], rows, gsem).wait()
        pltpu.sync_copy(rows, acc.at[dstv], add=True)
        return carry

    lax.fori_loop(0, NCHUNK, body, 0)
    plsc.subcore_barrier()
    pltpu.sync_copy(acc.at[pl.ds(s * STRIPE, STRIPE), :],
                    agg_hbm.at[pl.ds(c * NP + s * STRIPE, STRIPE), :])


def _mm_body(a_ref, w_ref, o_ref):
    o_ref[...] = jnp.dot(a_ref[...], w_ref[...],
                         preferred_element_type=jnp.float32)


def _scale_body(fx_ref, hs_ref, o_ref):
    sn = lax.rsqrt(hs_ref[0, :] + hs_ref[1, :] + 1.0)
    y = fx_ref[...] * sn[:, None]
    o_ref[...] = jnp.concatenate(
        [y, jnp.zeros((y.shape[0], DW - DH), jnp.float32)], axis=1)


def _l2_body(agg_ref, y1_ref, hs_ref, hd_ref, b1_ref, w2_ref, wf_ref, o_ref):
    a = (agg_ref[0] + agg_ref[1] + y1_ref[...])[:, :DH]
    dn = lax.rsqrt(hd_ref[0, :] + hd_ref[1, :] + 1.0)
    x2 = jnp.maximum(a * dn[:, None] + b1_ref[...][None, :], 0.0)
    sn = lax.rsqrt(hs_ref[0, :] + hs_ref[1, :] + 1.0)
    t = jnp.dot(x2 * sn[:, None], w2_ref[...],
                preferred_element_type=jnp.float32)
    m40 = jnp.dot(t, wf_ref[...], preferred_element_type=jnp.float32)
    o_ref[...] = jnp.concatenate(
        [m40, jnp.zeros((m40.shape[0], DW - DO), jnp.float32)], axis=1)


def _fin_body(agg_ref, m_ref, hd_ref, b2_ref, wf_ref, bf_ref, o_ref):
    a = (agg_ref[0] + agg_ref[1] + m_ref[...])[:, :DO]
    dn = lax.rsqrt(hd_ref[0, :] + hd_ref[1, :] + 1.0)
    bfin = jnp.dot(b2_ref[...][None, :], wf_ref[...],
                   preferred_element_type=jnp.float32) + bf_ref[...][None, :]
    o_ref[...] = a * dn[:, None] + bfin


_RB = 1024


def kernel(features, edge_index, W1, b1, W2, b2, Wf, bf):
    features_p = jnp.pad(features, ((0, NP - N), (0, 0)))
    pad = jnp.full((EPAD - E,), N, dtype=jnp.int32)
    src_p = jnp.concatenate([edge_index[0], pad])
    dst_p = jnp.concatenate([edge_index[1], pad])

    hs_flat, hd_flat = _hist_kernel(src_p, dst_p)
    hs = hs_flat.reshape(NCORES, NP)
    hd = hd_flat.reshape(NCORES, NP)
    zer = jnp.zeros((STRIPE, DW), jnp.float32)

    fx = pl.pallas_call(
        _mm_body,
        grid=(NP // _RB,),
        in_specs=[pl.BlockSpec((_RB, D_IN), lambda i: (i, 0)),
                  pl.BlockSpec((D_IN, DH), lambda i: (0, 0))],
        out_specs=pl.BlockSpec((_RB, DH), lambda i: (i, 0)),
        out_shape=jax.ShapeDtypeStruct((NP, DH), jnp.float32),
    )(features_p, W1)

    y1 = pl.pallas_call(
        _scale_body,
        grid=(NP // _RB,),
        in_specs=[pl.BlockSpec((_RB, DH), lambda i: (i, 0)),
                  pl.BlockSpec((NCORES, _RB), lambda i: (0, i))],
        out_specs=pl.BlockSpec((_RB, DW), lambda i: (i, 0)),
        out_shape=jax.ShapeDtypeStruct((NP, DW), jnp.float32),
    )(fx, hs)

    agg1 = _edge_pass(y1, src_p, dst_p, zer).reshape(NCORES, NP, DW)

    m = pl.pallas_call(
        _l2_body,
        grid=(NP // _RB,),
        in_specs=[pl.BlockSpec((NCORES, _RB, DW), lambda i: (0, i, 0)),
                  pl.BlockSpec((_RB, DW), lambda i: (i, 0)),
                  pl.BlockSpec((NCORES, _RB), lambda i: (0, i)),
                  pl.BlockSpec((NCORES, _RB), lambda i: (0, i)),
                  pl.BlockSpec((DH,), lambda i: (0,)),
                  pl.BlockSpec((DH, DH), lambda i: (0, 0)),
                  pl.BlockSpec((DH, DO), lambda i: (0, 0))],
        out_specs=pl.BlockSpec((_RB, DW), lambda i: (i, 0)),
        out_shape=jax.ShapeDtypeStruct((NP, DW), jnp.float32),
    )(agg1, y1, hs, hd, b1, W2, Wf)

    agg2 = _edge_pass(m, src_p, dst_p, zer).reshape(NCORES, NP, DW)

    out = pl.pallas_call(
        _fin_body,
        grid=(NP // _RB,),
        in_specs=[pl.BlockSpec((NCORES, _RB, DW), lambda i: (0, i, 0)),
                  pl.BlockSpec((_RB, DW), lambda i: (i, 0)),
                  pl.BlockSpec((NCORES, _RB), lambda i: (0, i)),
                  pl.BlockSpec((DH,), lambda i: (0,)),
                  pl.BlockSpec((DH, DO), lambda i: (0, 0)),
                  pl.BlockSpec((DO,), lambda i: (0,))],
        out_specs=pl.BlockSpec((_RB, DO), lambda i: (i, 0)),
        out_shape=jax.ShapeDtypeStruct((NP, DO), jnp.float32),
    )(agg2, m, hd, b2, Wf, bf)

    return out[:N]
